# baseline (speedup 1.0000x reference)
import jax
import jax.numpy as jnp
from jax import lax
from jax.experimental import pallas as pl
from jax.experimental.pallas import tpu as pltpu
from jax.experimental.pallas import tpu_sc as plsc

VOCAB = 1000000
EMB_DIM = 32
BATCH = 16384
FIELDS = 26

_info = plsc.get_sparse_core_info()
NC, NS = _info.num_cores, _info.num_subcores
NW = NC * NS

BLK = 128
NBLK = BATCH // BLK
UNITS = FIELDS * NBLK
PER_W = UNITS // NW
assert UNITS % NW == 0
assert PER_W % 2 == 0


def _body(ids_hbm, table_hbm, out_hbm,
          idx0, idx1, rows0, rows1, ot0, ot1,
          gsem0, gsem1, osem0, osem1):
    bufs = ((idx0, rows0, ot0, gsem0, osem0),
            (idx1, rows1, ot1, gsem1, osem1))
    wid = lax.axis_index("s") * NC + lax.axis_index("c")
    u0 = wid * PER_W

    def unit_fj(t):
        u = u0 + t
        f = u // NBLK
        j = u - f * NBLK
        return f, j

    def stage(t, b):
        idx, rows, _, gsem, _ = bufs[b]
        f, j = unit_fj(t)
        pltpu.sync_copy(ids_hbm.at[f, pl.ds(j * BLK, BLK)], idx)
        pltpu.async_copy(table_hbm.at[idx], rows, gsem)

    def wait_gather(b):
        idx, rows, _, gsem, _ = bufs[b]
        pltpu.make_async_copy(table_hbm.at[idx], rows, gsem).wait()

    def start_scatter(t, b):
        ot, osem = bufs[b][2], bufs[b][4]
        f, j = unit_fj(t)
        pltpu.async_copy(ot, out_hbm.at[f, :, pl.ds(j * BLK, BLK)], osem)

    def wait_scatter(t, b):
        ot, osem = bufs[b][2], bufs[b][4]
        f, j = unit_fj(t)
        pltpu.make_async_copy(ot, out_hbm.at[f, :, pl.ds(j * BLK, BLK)],
                              osem).wait()

    def extract(b):
        _, rows, ot, _, _ = bufs[b]
        iota = lax.iota(jnp.int32, 16)
        zero = iota * 0
        for g in range(BLK // 16):
            row_i = iota + (g * 16)
            for e in range(EMB_DIM):
                val = plsc.load_gather(rows, [row_i, zero + e])
                ot[e, pl.ds(g * 16, 16)] = val

    stage(0, 0)
    stage(1, 1)
    wait_gather(0)
    extract(0)
    start_scatter(0, 0)
    stage(2, 0)
    wait_gather(1)
    extract(1)
    start_scatter(1, 1)
    stage(3, 1)

    def pair(h):
        t = 2 * h
        wait_gather(0)
        wait_scatter(t - 2, 0)
        extract(0)
        start_scatter(t, 0)
        stage(t + 2, 0)
        wait_gather(1)
        wait_scatter(t - 1, 1)
        extract(1)
        start_scatter(t + 1, 1)
        stage(t + 3, 1)

    pl.loop(1, PER_W // 2 - 1)(pair)

    t = PER_W - 2
    wait_gather(0)
    wait_scatter(t - 2, 0)
    extract(0)
    start_scatter(t, 0)
    wait_gather(1)
    wait_scatter(t - 1, 1)
    extract(1)
    start_scatter(t + 1, 1)
    wait_scatter(PER_W - 2, 0)
    wait_scatter(PER_W - 1, 1)


def kernel(ids, table):
    ids_t = jnp.swapaxes(ids.astype(jnp.int32), 0, 1)

    mesh = plsc.VectorSubcoreMesh(core_axis_name="c", subcore_axis_name="s")
    out3 = pl.kernel(
        _body,
        out_type=jax.ShapeDtypeStruct((FIELDS, EMB_DIM, BATCH), jnp.float32),
        mesh=mesh,
        scratch_types=(
            [pltpu.VMEM((BLK,), jnp.int32)] * 2
            + [pltpu.VMEM((BLK, EMB_DIM), jnp.float32)] * 2
            + [pltpu.VMEM((EMB_DIM, BLK), jnp.float32)] * 2
            + [pltpu.SemaphoreType.DMA] * 4
        ),
        compiler_params=pltpu.CompilerParams(use_tc_tiling_on_sc=False,
                                             needs_layout_passes=False),
    )(ids_t, table)
    return jnp.transpose(out3, (2, 0, 1))


# speedup vs baseline: 1.2447x; 1.2447x over previous
import jax
import jax.numpy as jnp
from jax import lax
from jax.experimental import pallas as pl
from jax.experimental.pallas import tpu as pltpu
from jax.experimental.pallas import tpu_sc as plsc

VOCAB = 1000000
EMB_DIM = 32
BATCH = 16384
FIELDS = 26
TOTAL = BATCH * FIELDS

_info = plsc.get_sparse_core_info()
NC, NS = _info.num_cores, _info.num_subcores
NW = NC * NS

CHUNK = 416
PER_W = TOTAL // NW
NCHUNK = PER_W // CHUNK
NBUF = 8
K = 4

assert PER_W % CHUNK == 0
assert NCHUNK % NBUF == 0
assert CHUNK % 8 == 0
assert NCHUNK >= NBUF


def _body(ids_hbm, table_hbm, out_hbm, idx_v, rows_v, *sems):
    gsem, osem = sems[:NBUF], sems[NBUF:]
    wid = lax.axis_index("s") * NC + lax.axis_index("c")
    chunk0 = wid * NCHUNK

    pltpu.sync_copy(ids_hbm.at[pl.ds(chunk0, NCHUNK)], idx_v)

    def start_gather(c, b):
        pltpu.async_copy(table_hbm.at[idx_v.at[c]], rows_v.at[b], gsem[b])

    def wait_gather(c, b):
        pltpu.make_async_copy(table_hbm.at[idx_v.at[c]], rows_v.at[b],
                              gsem[b]).wait()

    def start_scatter(c, b):
        pltpu.async_copy(rows_v.at[b],
                         out_hbm.at[pl.ds((chunk0 + c) * CHUNK, CHUNK)],
                         osem[b])

    def wait_scatter(c, b):
        pltpu.make_async_copy(rows_v.at[b],
                              out_hbm.at[pl.ds((chunk0 + c) * CHUNK, CHUNK)],
                              osem[b]).wait()

    for b in range(K):
        start_gather(b, b)

    def group(g):
        for b in range(NBUF):
            c = g * NBUF + b
            wait_gather(c, b)
            start_scatter(c, b)
            nb = (b + K) % NBUF

            @pl.when(c - (NBUF - K) >= 0)
            def _():
                wait_scatter(c - (NBUF - K), nb)

            @pl.when(c + K <= NCHUNK - 1)
            def _():
                start_gather(c + K, nb)

    pl.loop(0, NCHUNK // NBUF)(group)

    for i in range(NBUF - K):
        c = NCHUNK - (NBUF - K) + i
        wait_scatter(c, c % NBUF)


def kernel(ids, table):
    flat_ids = ids.reshape(-1).astype(jnp.int32)
    ids2d = flat_ids.reshape(TOTAL // CHUNK, CHUNK)
    table = table.reshape(-1).reshape(VOCAB, EMB_DIM)

    mesh = plsc.VectorSubcoreMesh(core_axis_name="c", subcore_axis_name="s")
    out = pl.kernel(
        _body,
        out_type=jax.ShapeDtypeStruct((TOTAL, EMB_DIM), jnp.float32),
        mesh=mesh,
        scratch_types=(
            [pltpu.VMEM((NCHUNK, CHUNK), jnp.int32),
             pltpu.VMEM((NBUF, CHUNK, EMB_DIM), jnp.float32)]
            + [pltpu.SemaphoreType.DMA] * (2 * NBUF)
        ),
        compiler_params=pltpu.CompilerParams(use_tc_tiling_on_sc=False),
    )(ids2d, table)
    return out.reshape(-1).reshape(BATCH, FIELDS, EMB_DIM)
